# Initial kernel scaffold; baseline (speedup 1.0000x reference)
import jax
import jax.numpy as jnp
from jax.experimental import pallas as pl


def kernel(xs, W_ih, W_hh, b_ih, b_hh, W_out, b_out):
    raise NotImplementedError("write your pallas kernel here")


# speedup vs baseline: 2.3910x; 2.3910x over previous
import jax
import jax.numpy as jnp
from jax.experimental import pallas as pl
from jax.experimental.pallas import tpu as pltpu

SEQ_BLK = 64
B_BLK = 256


def _rnn_kernel(xs_ref, wih_ref, whh_ref, b_ref, wout_ref, bout_ref,
                out_ref, h_ref):
    j = pl.program_id(1)
    nseq = pl.num_programs(1)

    @pl.when(j == 0)
    def _():
        h_ref[...] = jnp.zeros_like(h_ref)

    wih = wih_ref[...]
    whh = whh_ref[...]
    b = b_ref[...]

    def step(t, h):
        x = xs_ref[t]
        z = (jnp.dot(x, wih, preferred_element_type=jnp.float32)
             + jnp.dot(h, whh, preferred_element_type=jnp.float32)
             + b)
        return jnp.tanh(z)

    h = jax.lax.fori_loop(0, SEQ_BLK, step, h_ref[...])
    h_ref[...] = h

    @pl.when(j == nseq - 1)
    def _():
        out_ref[...] = (jnp.sum(h * wout_ref[...], axis=1, keepdims=True)
                        + bout_ref[...])


def kernel(xs, W_ih, W_hh, b_ih, b_hh, W_out, b_out):
    seq, batch, inpt = xs.shape
    hidden = W_hh.shape[0]
    wih_t = W_ih.T
    whh_t = W_hh.T
    b = (b_ih + b_hh).reshape(1, hidden)
    wout = W_out.reshape(1, hidden)
    bout = b_out.reshape(1, 1)
    grid = (batch // B_BLK, seq // SEQ_BLK)
    return pl.pallas_call(
        _rnn_kernel,
        grid=grid,
        in_specs=[
            pl.BlockSpec((SEQ_BLK, B_BLK, inpt), lambda i, j: (j, i, 0)),
            pl.BlockSpec((inpt, hidden), lambda i, j: (0, 0)),
            pl.BlockSpec((hidden, hidden), lambda i, j: (0, 0)),
            pl.BlockSpec((1, hidden), lambda i, j: (0, 0)),
            pl.BlockSpec((1, hidden), lambda i, j: (0, 0)),
            pl.BlockSpec((1, 1), lambda i, j: (0, 0)),
        ],
        out_specs=pl.BlockSpec((B_BLK, 1), lambda i, j: (i, 0)),
        out_shape=jax.ShapeDtypeStruct((batch, 1), xs.dtype),
        scratch_shapes=[pltpu.VMEM((B_BLK, hidden), jnp.float32)],
        compiler_params=pltpu.CompilerParams(
            dimension_semantics=("parallel", "arbitrary"),
        ),
    )(xs, wih_t, whh_t, b, wout, bout)
